# baseline (speedup 1.0000x reference)
import functools

import jax
import jax.numpy as jnp
from jax import lax
from jax.experimental import pallas as pl
from jax.experimental.pallas import tpu as pltpu
from jax.experimental.pallas import tpu_sc as plsc

N = 320000
D = 128
S = 10000

NW = 16
S_PAD = 10240
SEG_PER_W = S_PAD // NW
N_PAD = 327680
ATOMS_PER_W = N_PAD // NW
CHUNK = 128
CHUNKS_PER_W = ATOMS_PER_W // CHUNK

ROWS_BLK = 2560
GRID_A = N // ROWS_BLK


def _rowmean_body(x_ref, o_ref):
    o_ref[0, 0, :] = jnp.sum(x_ref[...], axis=1) * (1.0 / D)


def _rowmean(x):
    return pl.pallas_call(
        _rowmean_body,
        grid=(GRID_A,),
        in_specs=[pl.BlockSpec((ROWS_BLK, D), lambda i: (i, 0))],
        out_specs=pl.BlockSpec((1, 1, ROWS_BLK), lambda i: (i, 0, 0)),
        out_shape=jax.ShapeDtypeStruct((GRID_A, 1, ROWS_BLK), jnp.float32),
    )(x)


def _segsum_body(vals_hbm, ids_hbm, na_hbm, out_hbm,
                 vals_v, ids_v, acc_sh, seg_v, na_v, out_v):
    wid = lax.axis_index("s")
    row0 = wid * CHUNKS_PER_W
    sbase = wid * SEG_PER_W

    pltpu.sync_copy(vals_hbm.at[pl.ds(row0, CHUNKS_PER_W)], vals_v)
    pltpu.sync_copy(ids_hbm.at[pl.ds(row0, CHUNKS_PER_W)], ids_v)

    def _zero(i, _):
        seg_v[pl.ds(i * 16, 16)] = jnp.zeros((16,), jnp.float32)
        return 0
    lax.fori_loop(0, SEG_PER_W // 16, _zero, 0)
    pltpu.sync_copy(seg_v, acc_sh.at[pl.ds(sbase, SEG_PER_W)])
    plsc.subcore_barrier()

    def _scatter(j, _):
        pltpu.sync_copy(vals_v.at[j], acc_sh.at[ids_v.at[j]], add=True)
        return 0
    lax.fori_loop(0, CHUNKS_PER_W, _scatter, 0)
    plsc.subcore_barrier()

    pltpu.sync_copy(acc_sh.at[pl.ds(sbase, SEG_PER_W)], seg_v)
    pltpu.sync_copy(na_hbm.at[pl.ds(sbase, SEG_PER_W)], na_v)

    def _div(i, _):
        sl = pl.ds(i * 16, 16)
        out_v[sl] = seg_v[sl] / na_v[sl]
        return 0
    lax.fori_loop(0, SEG_PER_W // 16, _div, 0)
    pltpu.sync_copy(out_v, out_hbm.at[pl.ds(sbase, SEG_PER_W)])


@functools.partial(jax.jit, static_argnames=())
def _segsum(vals2d, ids2d, na_pad):
    mesh = plsc.VectorSubcoreMesh(
        core_axis_name="c", subcore_axis_name="s", num_cores=1)
    f = pl.kernel(
        _segsum_body,
        out_type=jax.ShapeDtypeStruct((S_PAD,), jnp.float32),
        mesh=mesh,
        scratch_types=[
            pltpu.VMEM((CHUNKS_PER_W, CHUNK), jnp.float32),
            pltpu.VMEM((CHUNKS_PER_W, CHUNK), jnp.int32),
            pltpu.VMEM_SHARED((S_PAD,), jnp.float32),
            pltpu.VMEM((SEG_PER_W,), jnp.float32),
            pltpu.VMEM((SEG_PER_W,), jnp.float32),
            pltpu.VMEM((SEG_PER_W,), jnp.float32),
        ],
    )
    return f(vals2d, ids2d, na_pad)


def kernel(atom_features, segment_ids, num_atoms):
    rm = _rowmean(atom_features).reshape(N)
    rm2d = jnp.pad(rm, (0, N_PAD - N)).reshape(N_PAD // CHUNK, CHUNK)
    ids2d = jnp.pad(segment_ids, (0, N_PAD - N)).reshape(N_PAD // CHUNK, CHUNK)
    na_pad = jnp.pad(num_atoms, (0, S_PAD - S), constant_values=1.0)
    out = _segsum(rm2d, ids2d, na_pad)
    return out[:S].reshape(S, 1)


# speedup vs baseline: 1.3105x; 1.3105x over previous
import functools

import jax
import jax.numpy as jnp
from jax import lax
from jax.experimental import pallas as pl
from jax.experimental.pallas import tpu as pltpu
from jax.experimental.pallas import tpu_sc as plsc

N = 320000
D = 128
S = 10000

NW = 16
S_PAD = 10240
SEG_PER_W = S_PAD // NW
N_PAD = 327680
ATOMS_PER_W = N_PAD // NW
CHUNK = 128
CHUNKS_PER_W = ATOMS_PER_W // CHUNK

ROWS_BLK = 2560
GRID_A = N // ROWS_BLK


def _rowmean_body(x_ref, o_ref):
    ones = jnp.full((8, D), 1.0 / D, dtype=jnp.float32)
    r = jax.lax.dot_general(ones, x_ref[...], (((1,), (1,)), ((), ())),
                            preferred_element_type=jnp.float32)
    o_ref[0, :, :] = r


def _rowmean(x):
    return pl.pallas_call(
        _rowmean_body,
        grid=(GRID_A,),
        in_specs=[pl.BlockSpec((ROWS_BLK, D), lambda i: (i, 0))],
        out_specs=pl.BlockSpec((1, 8, ROWS_BLK), lambda i: (i, 0, 0)),
        out_shape=jax.ShapeDtypeStruct((GRID_A, 8, ROWS_BLK), jnp.float32),
    )(x)


def _segsum_body(vals_hbm, ids_hbm, na_hbm, out_hbm,
                 vals_v, ids_v, acc_sh, seg_v, na_v, out_v):
    wid = lax.axis_index("s")
    row0 = wid * CHUNKS_PER_W
    sbase = wid * SEG_PER_W

    pltpu.sync_copy(vals_hbm.at[pl.ds(row0, CHUNKS_PER_W)], vals_v)
    pltpu.sync_copy(ids_hbm.at[pl.ds(row0, CHUNKS_PER_W)], ids_v)

    def _zero(i, _):
        seg_v[pl.ds(i * 16, 16)] = jnp.zeros((16,), jnp.float32)
        return 0
    lax.fori_loop(0, SEG_PER_W // 16, _zero, 0)
    pltpu.sync_copy(seg_v, acc_sh.at[pl.ds(sbase, SEG_PER_W)])
    plsc.subcore_barrier()

    def _scatter(j, _):
        pltpu.sync_copy(vals_v.at[j], acc_sh.at[ids_v.at[j]], add=True)
        return 0
    lax.fori_loop(0, CHUNKS_PER_W, _scatter, 0)
    plsc.subcore_barrier()

    pltpu.sync_copy(acc_sh.at[pl.ds(sbase, SEG_PER_W)], seg_v)
    pltpu.sync_copy(na_hbm.at[pl.ds(sbase, SEG_PER_W)], na_v)

    def _div(i, _):
        sl = pl.ds(i * 16, 16)
        out_v[sl] = seg_v[sl] / na_v[sl]
        return 0
    lax.fori_loop(0, SEG_PER_W // 16, _div, 0)
    pltpu.sync_copy(out_v, out_hbm.at[pl.ds(sbase, SEG_PER_W)])


@functools.partial(jax.jit, static_argnames=())
def _segsum(vals2d, ids2d, na_pad):
    mesh = plsc.VectorSubcoreMesh(
        core_axis_name="c", subcore_axis_name="s", num_cores=1)
    f = pl.kernel(
        _segsum_body,
        out_type=jax.ShapeDtypeStruct((S_PAD,), jnp.float32),
        mesh=mesh,
        scratch_types=[
            pltpu.VMEM((CHUNKS_PER_W, CHUNK), jnp.float32),
            pltpu.VMEM((CHUNKS_PER_W, CHUNK), jnp.int32),
            pltpu.VMEM_SHARED((S_PAD,), jnp.float32),
            pltpu.VMEM((SEG_PER_W,), jnp.float32),
            pltpu.VMEM((SEG_PER_W,), jnp.float32),
            pltpu.VMEM((SEG_PER_W,), jnp.float32),
        ],
    )
    return f(vals2d, ids2d, na_pad)


def kernel(atom_features, segment_ids, num_atoms):
    rm = _rowmean(atom_features)[:, 0, :].reshape(N)
    rm2d = jnp.pad(rm, (0, N_PAD - N)).reshape(N_PAD // CHUNK, CHUNK)
    ids2d = jnp.pad(segment_ids, (0, N_PAD - N)).reshape(N_PAD // CHUNK, CHUNK)
    na_pad = jnp.pad(num_atoms, (0, S_PAD - S), constant_values=1.0)
    out = _segsum(rm2d, ids2d, na_pad)
    return out[:S].reshape(S, 1)


# speedup vs baseline: 1.4047x; 1.0719x over previous
import jax
import jax.numpy as jnp
from jax import lax
from jax.experimental import pallas as pl
from jax.experimental.pallas import tpu as pltpu
from jax.experimental.pallas import tpu_sc as plsc

N = 320000
D = 128
S = 10000

NW = 16
NROW = N // D
ROWS_PER_W = 160
ROWS_LAST = NROW - (NW - 1) * ROWS_PER_W
SEG_PER_W = 640
SEG_LAST = S - (NW - 1) * SEG_PER_W
S_PAD = NW * SEG_PER_W

ROWS_BLK = 2560
GRID_A = N // ROWS_BLK
BLKS_PER_W = ROWS_PER_W * D // ROWS_BLK
BLKS_LAST = ROWS_LAST * D // ROWS_BLK
RPB = ROWS_BLK // D


def _rowmean_body(x_ref, o_ref):
    ones = jnp.full((8, D), 1.0 / D, dtype=jnp.float32)
    r = jax.lax.dot_general(ones, x_ref[...], (((1,), (1,)), ((), ())),
                            preferred_element_type=jnp.float32)
    o_ref[0, 0, :] = r[0, :]


def _rowmean(x):
    return pl.pallas_call(
        _rowmean_body,
        grid=(GRID_A,),
        in_specs=[pl.BlockSpec((ROWS_BLK, D), lambda i: (i, 0))],
        out_specs=pl.BlockSpec((1, 1, ROWS_BLK), lambda i: (i, 0, 0)),
        out_shape=jax.ShapeDtypeStruct((GRID_A, 1, ROWS_BLK), jnp.float32),
    )(x)


def _segsum_body(vals_hbm, ids_hbm, na_hbm, out_hbm,
                 vals_v, ids_v, acc_sh, seg_v, na_v, out_v):
    wid = lax.axis_index("s")
    b0 = wid * BLKS_PER_W
    row0 = wid * ROWS_PER_W
    sbase = wid * SEG_PER_W
    last = NW - 1

    @pl.when(wid < last)
    def _():
        def _ld(t, _):
            pltpu.sync_copy(vals_hbm.at[b0 + t, 0], vals_v.at[t])
            return 0
        lax.fori_loop(0, BLKS_PER_W, _ld, 0)
        pltpu.sync_copy(ids_hbm.at[pl.ds(row0, ROWS_PER_W)], ids_v)

    @pl.when(wid == last)
    def _():
        def _ld(t, _):
            pltpu.sync_copy(vals_hbm.at[b0 + t, 0], vals_v.at[t])
            return 0
        lax.fori_loop(0, BLKS_LAST, _ld, 0)
        pltpu.sync_copy(ids_hbm.at[pl.ds(last * ROWS_PER_W, ROWS_LAST)],
                        ids_v.at[pl.ds(0, ROWS_LAST)])

    def _zero(i, _):
        seg_v[pl.ds(i * 16, 16)] = jnp.zeros((16,), jnp.float32)
        return 0
    lax.fori_loop(0, SEG_PER_W // 16, _zero, 0)
    pltpu.sync_copy(seg_v, acc_sh.at[pl.ds(sbase, SEG_PER_W)])
    plsc.subcore_barrier()

    nrows = jnp.where(wid < last, ROWS_PER_W, ROWS_LAST)

    def _scatter(j, _):
        t = j // RPB
        c = j % RPB
        pltpu.sync_copy(vals_v.at[t, pl.ds(c * D, D)],
                        acc_sh.at[ids_v.at[j]], add=True)
        return 0
    lax.fori_loop(0, nrows, _scatter, 0)
    plsc.subcore_barrier()

    pltpu.sync_copy(acc_sh.at[pl.ds(sbase, SEG_PER_W)], seg_v)

    @pl.when(wid < last)
    def _():
        pltpu.sync_copy(na_hbm.at[pl.ds(sbase, SEG_PER_W)], na_v)

    @pl.when(wid == last)
    def _():
        pltpu.sync_copy(na_hbm.at[pl.ds(last * SEG_PER_W, SEG_LAST)],
                        na_v.at[pl.ds(0, SEG_LAST)])

    def _div(i, _):
        sl = pl.ds(i * 16, 16)
        out_v[sl] = seg_v[sl] / na_v[sl]
        return 0
    lax.fori_loop(0, SEG_PER_W // 16, _div, 0)

    @pl.when(wid < last)
    def _():
        pltpu.sync_copy(out_v, out_hbm.at[pl.ds(sbase, SEG_PER_W)])

    @pl.when(wid == last)
    def _():
        pltpu.sync_copy(out_v.at[pl.ds(0, SEG_LAST)],
                        out_hbm.at[pl.ds(last * SEG_PER_W, SEG_LAST)])


def _segsum(vals3d, ids2d, na):
    mesh = plsc.VectorSubcoreMesh(
        core_axis_name="c", subcore_axis_name="s", num_cores=1)
    f = pl.kernel(
        _segsum_body,
        out_type=jax.ShapeDtypeStruct((S,), jnp.float32),
        mesh=mesh,
        scratch_types=[
            pltpu.VMEM((BLKS_PER_W, ROWS_BLK), jnp.float32),
            pltpu.VMEM((ROWS_PER_W, D), jnp.int32),
            pltpu.VMEM_SHARED((S_PAD,), jnp.float32),
            pltpu.VMEM((SEG_PER_W,), jnp.float32),
            pltpu.VMEM((SEG_PER_W,), jnp.float32),
            pltpu.VMEM((SEG_PER_W,), jnp.float32),
        ],
    )
    return f(vals3d, ids2d, na)


def kernel(atom_features, segment_ids, num_atoms):
    rm3d = _rowmean(atom_features)
    ids2d = segment_ids.reshape(NROW, D)
    out = _segsum(rm3d, ids2d, num_atoms)
    return out.reshape(S, 1)


# speedup vs baseline: 2.2761x; 1.6203x over previous
import jax
import jax.numpy as jnp
from jax import lax
from jax.experimental import pallas as pl
from jax.experimental.pallas import tpu as pltpu
from jax.experimental.pallas import tpu_sc as plsc

N = 320000
D = 128
S = 10000

NROW = N // D
ROWS_BLK = 2560
TC_BLKS = 61
TC_ROWS = TC_BLKS * ROWS_BLK // D
SC_ROWS = NROW - TC_ROWS
RPT = SC_ROWS // 32
CRW = 2
NCH = RPT // CRW
CW = CRW * D * D
CA = CRW * D

SEG_PER_W = 640
SEG_LAST = S - 15 * SEG_PER_W
S_PAD = 16 * SEG_PER_W

RPC = TC_ROWS // 2
R_HI = 39
R_LO = 38


def _rowmean_body(x_ref, o_ref):
    ones = jnp.full((8, D), 1.0 / D, dtype=jnp.float32)
    r = jax.lax.dot_general(ones, x_ref[...], (((1,), (1,)), ((), ())),
                            preferred_element_type=jnp.float32)
    o_ref[0, 0, :] = r[0, :]


def _rowmean(x):
    return pl.pallas_call(
        _rowmean_body,
        grid=(TC_BLKS,),
        in_specs=[pl.BlockSpec((ROWS_BLK, D), lambda i: (i, 0))],
        out_specs=pl.BlockSpec((1, 1, ROWS_BLK), lambda i: (i, 0, 0)),
        out_shape=jax.ShapeDtypeStruct((TC_BLKS, 1, ROWS_BLK), jnp.float32),
    )(x)


def _reduce_chunk(buf, rs_v):
    lane = lax.iota(jnp.int32, 16)
    mask0 = lane == 0
    dnums = lax.GatherDimensionNumbers(
        offset_dims=(), collapsed_slice_dims=(0,), start_index_map=(0,))

    def _perm(v, idx):
        return lax.gather(v, idx[:, None], dnums, (1,),
                          mode=lax.GatherScatterMode.PROMISE_IN_BOUNDS)

    def _grp(g, _):
        base = g * 16 * D
        w = jnp.zeros((16,), jnp.float32)
        for a in range(16):
            v = buf[pl.ds(base + a * D, 16)]
            for k in range(1, 8):
                v = v + buf[pl.ds(base + a * D + k * 16, 16)]
            for k in (8, 4, 2, 1):
                v = v + _perm(v, lane ^ k)
            w = jnp.where(lane == a, v, w)
        rs_v[pl.ds(g * 16, 16)] = w * (1.0 / D)
        return 0
    lax.fori_loop(0, CA // 16, _grp, 0)


def _scpartial_body(x_hbm, ids_hbm, part_hbm,
                    buf0, buf1, rs_v, ids_v, acc_sh, z_v, sem):
    cid = lax.axis_index("c")
    sid = lax.axis_index("s")
    tile = cid * 16 + sid
    row0 = TC_ROWS + tile * RPT
    base = row0 * D * D

    pltpu.async_copy(x_hbm.at[pl.ds(base, CW)], buf0, sem)
    pltpu.sync_copy(ids_hbm.at[pl.ds(row0 - 4, RPT + 8)], ids_v)

    def _zero(i, _):
        z_v[pl.ds(i * 16, 16)] = jnp.zeros((16,), jnp.float32)
        return 0
    lax.fori_loop(0, SEG_PER_W // 16, _zero, 0)
    pltpu.sync_copy(z_v, acc_sh.at[pl.ds(sid * SEG_PER_W, SEG_PER_W)])
    plsc.subcore_barrier()

    def _scatter(c):
        for m in range(CRW):
            pltpu.sync_copy(rs_v.at[pl.ds(m * D, D)],
                            acc_sh.at[ids_v.at[4 + c * CRW + m]], add=True)

    def _loop(c, _):
        @pl.when(c % 2 == 0)
        def _():
            pltpu.make_async_copy(x_hbm.at[pl.ds(base, CW)], buf0, sem).wait()
            @pl.when(c + 1 < NCH)
            def _():
                pltpu.async_copy(
                    x_hbm.at[pl.ds(base + (c + 1) * CW, CW)], buf1, sem)
            _reduce_chunk(buf0, rs_v)
            _scatter(c)

        @pl.when(c % 2 == 1)
        def _():
            pltpu.make_async_copy(x_hbm.at[pl.ds(base, CW)], buf1, sem).wait()
            @pl.when(c + 1 < NCH)
            def _():
                pltpu.async_copy(
                    x_hbm.at[pl.ds(base + (c + 1) * CW, CW)], buf0, sem)
            _reduce_chunk(buf1, rs_v)
            _scatter(c)
        return 0

    lax.fori_loop(0, NCH, _loop, 0)
    plsc.subcore_barrier()
    pltpu.sync_copy(acc_sh.at[pl.ds(sid * SEG_PER_W, SEG_PER_W)],
                    part_hbm.at[cid, pl.ds(sid * SEG_PER_W, SEG_PER_W)])


def _scpartial(xflat, ids2d):
    mesh = plsc.VectorSubcoreMesh(core_axis_name="c", subcore_axis_name="s")
    f = pl.kernel(
        _scpartial_body,
        out_type=jax.ShapeDtypeStruct((2, S_PAD), jnp.float32),
        mesh=mesh,
        scratch_types=[
            pltpu.VMEM((CW,), jnp.float32),
            pltpu.VMEM((CW,), jnp.float32),
            pltpu.VMEM((CA + 16,), jnp.float32),
            pltpu.VMEM((RPT + 8, D), jnp.int32),
            pltpu.VMEM_SHARED((S_PAD,), jnp.float32),
            pltpu.VMEM((SEG_PER_W,), jnp.float32),
            pltpu.SemaphoreType.DMA,
        ],
    )
    return f(xflat, ids2d)


def _scfinal_body(rm_hbm, ids_hbm, part_hbm, na_hbm, out_hbm,
                  vals_v, ids_v, acc_sh, seg_v, na_v, out_v):
    cid = lax.axis_index("c")
    sid = lax.axis_index("s")

    pltpu.sync_copy(part_hbm.at[cid, pl.ds(sid * SEG_PER_W, SEG_PER_W)],
                    acc_sh.at[pl.ds(sid * SEG_PER_W, SEG_PER_W)])
    plsc.subcore_barrier()

    r0 = cid * RPC + R_LO * sid + jnp.minimum(sid, 2)
    r0a = (r0 // 8) * 8
    off = r0 - r0a

    @pl.when(sid < 2)
    def _():
        pltpu.sync_copy(rm_hbm.at[pl.ds(r0 * D, R_HI * D)],
                        vals_v.at[pl.ds(0, R_HI * D)])

    @pl.when(sid >= 2)
    def _():
        pltpu.sync_copy(rm_hbm.at[pl.ds(r0 * D, R_LO * D)],
                        vals_v.at[pl.ds(0, R_LO * D)])

    pltpu.sync_copy(ids_hbm.at[pl.ds(r0a, 48)], ids_v)

    nr = jnp.where(sid < 2, R_HI, R_LO)

    def _scatter(j, _):
        pltpu.sync_copy(vals_v.at[pl.ds(j * D, D)],
                        acc_sh.at[ids_v.at[off + j]], add=True)
        return 0
    lax.fori_loop(0, nr, _scatter, 0)
    plsc.subcore_barrier()

    sbase = sid * SEG_PER_W
    pltpu.sync_copy(acc_sh.at[pl.ds(sbase, SEG_PER_W)], seg_v)

    @pl.when(sid < 15)
    def _():
        pltpu.sync_copy(na_hbm.at[pl.ds(sbase, SEG_PER_W)], na_v)

    @pl.when(sid == 15)
    def _():
        pltpu.sync_copy(na_hbm.at[pl.ds(15 * SEG_PER_W, SEG_LAST)],
                        na_v.at[pl.ds(0, SEG_LAST)])

    def _div(i, _):
        sl = pl.ds(i * 16, 16)
        out_v[sl] = seg_v[sl] / na_v[sl]
        return 0
    lax.fori_loop(0, SEG_PER_W // 16, _div, 0)
    pltpu.sync_copy(out_v, out_hbm.at[cid, pl.ds(sbase, SEG_PER_W)])


def _scfinal(rm_flat, ids2d, part, na):
    mesh = plsc.VectorSubcoreMesh(core_axis_name="c", subcore_axis_name="s")
    f = pl.kernel(
        _scfinal_body,
        out_type=jax.ShapeDtypeStruct((2, S_PAD), jnp.float32),
        mesh=mesh,
        scratch_types=[
            pltpu.VMEM((R_HI * D,), jnp.float32),
            pltpu.VMEM((48, D), jnp.int32),
            pltpu.VMEM_SHARED((S_PAD,), jnp.float32),
            pltpu.VMEM((SEG_PER_W,), jnp.float32),
            pltpu.VMEM((SEG_PER_W,), jnp.float32),
            pltpu.VMEM((SEG_PER_W,), jnp.float32),
        ],
    )
    return f(rm_flat, ids2d, part, na)


def kernel(atom_features, segment_ids, num_atoms):
    rm3d = _rowmean(atom_features)
    xflat = atom_features.reshape(N * D)
    ids2d = segment_ids.reshape(NROW, D)
    part = _scpartial(xflat, ids2d)
    out2 = _scfinal(rm3d.reshape(TC_ROWS * D), ids2d, part, num_atoms)
    return (out2[0, :S] + out2[1, :S]).reshape(S, 1)


# speedup vs baseline: 2.2785x; 1.0010x over previous
import jax
import jax.numpy as jnp
from jax import lax
from jax.experimental import pallas as pl
from jax.experimental.pallas import tpu as pltpu
from jax.experimental.pallas import tpu_sc as plsc

N = 320000
D = 128
S = 10000

NROW = N // D
ROWS_BLK = 2560
TC_BLKS = 45
TC_ROWS = TC_BLKS * ROWS_BLK // D
SC_ROWS = NROW - TC_ROWS
RPT = SC_ROWS // 32
CRW = 2
NCH = RPT // CRW
CW = CRW * D * D
CA = CRW * D

SEG_PER_W = 640
SEG_LAST = S - 15 * SEG_PER_W
S_PAD = 16 * SEG_PER_W

RPC = TC_ROWS // 2
R_HI = 29
R_LO = 28


def _rowmean_body(x_ref, o_ref):
    ones = jnp.full((8, D), 1.0 / D, dtype=jnp.float32)
    r = jax.lax.dot_general(ones, x_ref[...], (((1,), (1,)), ((), ())),
                            preferred_element_type=jnp.float32)
    o_ref[0, 0, :] = r[0, :]


def _rowmean(x):
    return pl.pallas_call(
        _rowmean_body,
        grid=(TC_BLKS,),
        in_specs=[pl.BlockSpec((ROWS_BLK, D), lambda i: (i, 0))],
        out_specs=pl.BlockSpec((1, 1, ROWS_BLK), lambda i: (i, 0, 0)),
        out_shape=jax.ShapeDtypeStruct((TC_BLKS, 1, ROWS_BLK), jnp.float32),
    )(x)


def _reduce_chunk(buf, rs_v):
    lane = lax.iota(jnp.int32, 16)
    mask0 = lane == 0
    dnums = lax.GatherDimensionNumbers(
        offset_dims=(), collapsed_slice_dims=(0,), start_index_map=(0,))

    def _perm(v, idx):
        return lax.gather(v, idx[:, None], dnums, (1,),
                          mode=lax.GatherScatterMode.PROMISE_IN_BOUNDS)

    def _grp(g, _):
        base = g * 16 * D
        w = jnp.zeros((16,), jnp.float32)
        for a in range(16):
            v = buf[pl.ds(base + a * D, 16)]
            for k in range(1, 8):
                v = v + buf[pl.ds(base + a * D + k * 16, 16)]
            for k in (8, 4, 2, 1):
                v = v + _perm(v, lane ^ k)
            w = jnp.where(lane == a, v, w)
        rs_v[pl.ds(g * 16, 16)] = w * (1.0 / D)
        return 0
    lax.fori_loop(0, CA // 16, _grp, 0)


def _scpartial_body(x_hbm, ids_hbm, part_hbm,
                    buf0, buf1, rs_v, ids_v, acc_sh, z_v, sem):
    cid = lax.axis_index("c")
    sid = lax.axis_index("s")
    tile = cid * 16 + sid
    row0 = TC_ROWS + tile * RPT
    base = row0 * D * D

    pltpu.async_copy(x_hbm.at[pl.ds(base, CW)], buf0, sem)
    pltpu.sync_copy(ids_hbm.at[pl.ds(row0 * D, RPT * D)], ids_v)

    def _zero(i, _):
        z_v[pl.ds(i * 16, 16)] = jnp.zeros((16,), jnp.float32)
        return 0
    lax.fori_loop(0, SEG_PER_W // 16, _zero, 0)
    pltpu.sync_copy(z_v, acc_sh.at[pl.ds(sid * SEG_PER_W, SEG_PER_W)])
    plsc.subcore_barrier()

    def _scatter(c):
        for m in range(CRW):
            pltpu.sync_copy(rs_v.at[pl.ds(m * D, D)],
                            acc_sh.at[ids_v.at[pl.ds((c * CRW + m) * D, D)]],
                            add=True)

    def _loop(c, _):
        @pl.when(c % 2 == 0)
        def _():
            pltpu.make_async_copy(x_hbm.at[pl.ds(base, CW)], buf0, sem).wait()
            @pl.when(c + 1 < NCH)
            def _():
                pltpu.async_copy(
                    x_hbm.at[pl.ds(base + (c + 1) * CW, CW)], buf1, sem)
            _reduce_chunk(buf0, rs_v)
            _scatter(c)

        @pl.when(c % 2 == 1)
        def _():
            pltpu.make_async_copy(x_hbm.at[pl.ds(base, CW)], buf1, sem).wait()
            @pl.when(c + 1 < NCH)
            def _():
                pltpu.async_copy(
                    x_hbm.at[pl.ds(base + (c + 1) * CW, CW)], buf0, sem)
            _reduce_chunk(buf1, rs_v)
            _scatter(c)
        return 0

    lax.fori_loop(0, NCH, _loop, 0)
    plsc.subcore_barrier()
    pltpu.sync_copy(acc_sh.at[pl.ds(sid * SEG_PER_W, SEG_PER_W)],
                    part_hbm.at[cid, pl.ds(sid * SEG_PER_W, SEG_PER_W)])


def _scpartial(xflat, ids2d):
    mesh = plsc.VectorSubcoreMesh(core_axis_name="c", subcore_axis_name="s")
    f = pl.kernel(
        _scpartial_body,
        out_type=jax.ShapeDtypeStruct((2, S_PAD), jnp.float32),
        mesh=mesh,
        scratch_types=[
            pltpu.VMEM((CW,), jnp.float32),
            pltpu.VMEM((CW,), jnp.float32),
            pltpu.VMEM((CA + 16,), jnp.float32),
            pltpu.VMEM((RPT * D,), jnp.int32),
            pltpu.VMEM_SHARED((S_PAD,), jnp.float32),
            pltpu.VMEM((SEG_PER_W,), jnp.float32),
            pltpu.SemaphoreType.DMA,
        ],
    )
    return f(xflat, ids2d)


def _scfinal_body(rm_hbm, ids_hbm, part_hbm, na_hbm, out_hbm,
                  vals_v, ids_v, acc_sh, seg_v, na_v, out_v):
    cid = lax.axis_index("c")
    sid = lax.axis_index("s")

    pltpu.sync_copy(part_hbm.at[cid, pl.ds(sid * SEG_PER_W, SEG_PER_W)],
                    acc_sh.at[pl.ds(sid * SEG_PER_W, SEG_PER_W)])
    plsc.subcore_barrier()

    r0 = cid * RPC + R_LO * sid + jnp.minimum(sid, 2)
    @pl.when(sid < 2)
    def _():
        pltpu.sync_copy(rm_hbm.at[pl.ds(r0 * D, R_HI * D)],
                        vals_v.at[pl.ds(0, R_HI * D)])
        pltpu.sync_copy(ids_hbm.at[pl.ds(r0 * D, R_HI * D)],
                        ids_v.at[pl.ds(0, R_HI * D)])

    @pl.when(sid >= 2)
    def _():
        pltpu.sync_copy(rm_hbm.at[pl.ds(r0 * D, R_LO * D)],
                        vals_v.at[pl.ds(0, R_LO * D)])
        pltpu.sync_copy(ids_hbm.at[pl.ds(r0 * D, R_LO * D)],
                        ids_v.at[pl.ds(0, R_LO * D)])

    nr = jnp.where(sid < 2, R_HI, R_LO)

    def _scatter(j, _):
        pltpu.sync_copy(vals_v.at[pl.ds(j * D, D)],
                        acc_sh.at[ids_v.at[pl.ds(j * D, D)]], add=True)
        return 0
    lax.fori_loop(0, nr, _scatter, 0)
    plsc.subcore_barrier()

    sbase = sid * SEG_PER_W
    pltpu.sync_copy(acc_sh.at[pl.ds(sbase, SEG_PER_W)], seg_v)

    @pl.when(sid < 15)
    def _():
        pltpu.sync_copy(na_hbm.at[pl.ds(sbase, SEG_PER_W)], na_v)

    @pl.when(sid == 15)
    def _():
        pltpu.sync_copy(na_hbm.at[pl.ds(15 * SEG_PER_W, SEG_LAST)],
                        na_v.at[pl.ds(0, SEG_LAST)])

    def _div(i, _):
        sl = pl.ds(i * 16, 16)
        out_v[sl] = seg_v[sl] / na_v[sl]
        return 0
    lax.fori_loop(0, SEG_PER_W // 16, _div, 0)
    pltpu.sync_copy(out_v, out_hbm.at[cid, pl.ds(sbase, SEG_PER_W)])


def _scfinal(rm_flat, ids2d, part, na):
    mesh = plsc.VectorSubcoreMesh(core_axis_name="c", subcore_axis_name="s")
    f = pl.kernel(
        _scfinal_body,
        out_type=jax.ShapeDtypeStruct((2, S_PAD), jnp.float32),
        mesh=mesh,
        scratch_types=[
            pltpu.VMEM((R_HI * D,), jnp.float32),
            pltpu.VMEM((R_HI * D,), jnp.int32),
            pltpu.VMEM_SHARED((S_PAD,), jnp.float32),
            pltpu.VMEM((SEG_PER_W,), jnp.float32),
            pltpu.VMEM((SEG_PER_W,), jnp.float32),
            pltpu.VMEM((SEG_PER_W,), jnp.float32),
        ],
    )
    return f(rm_flat, ids2d, part, na)


def kernel(atom_features, segment_ids, num_atoms):
    rm3d = _rowmean(atom_features)
    xflat = atom_features.reshape(N * D)
    part = _scpartial(xflat, segment_ids)
    out2 = _scfinal(rm3d.reshape(TC_ROWS * D), segment_ids, part, num_atoms)
    return (out2[0, :S] + out2[1, :S]).reshape(S, 1)
